# Initial kernel scaffold; baseline (speedup 1.0000x reference)
import jax
import jax.numpy as jnp
from jax.experimental import pallas as pl


def kernel(xs, ys, e_times, hs, h_times, neg_node, h_times_mask, emb_table, delta_table):
    raise NotImplementedError("write your pallas kernel here")


# speedup vs baseline: 1.3706x; 1.3706x over previous
import functools

import jax
import jax.numpy as jnp
from jax import lax
from jax.experimental import pallas as pl
from jax.experimental.pallas import tpu as pltpu
from jax.experimental.pallas import tpu_sc as plsc

NODE = 1000000
D = 64
B = 16384
H = 20
N = 5

_info = plsc.get_sparse_core_info()
_NC, _NS = _info.num_cores, _info.num_subcores
NW = _NC * _NS
BPW = B // NW

_sc_mesh = plsc.VectorSubcoreMesh(core_axis_name="c", subcore_axis_name="s")


@functools.partial(
    pl.kernel,
    mesh=_sc_mesh,
    compiler_params=pltpu.CompilerParams(use_tc_tiling_on_sc=False),
    out_type=[
        jax.ShapeDtypeStruct((B, D), jnp.float32),
        jax.ShapeDtypeStruct((B, D), jnp.float32),
        jax.ShapeDtypeStruct((H, B, D), jnp.float32),
        jax.ShapeDtypeStruct((N, B, D), jnp.float32),
        jax.ShapeDtypeStruct((B,), jnp.float32),
    ],
    scratch_types=[
        pltpu.VMEM((BPW,), jnp.int32),
        pltpu.VMEM((BPW, D), jnp.float32),
        pltpu.VMEM((BPW,), jnp.float32),
        pltpu.SemaphoreType.DMA,
    ],
)
def _sc_gather(table, dflat, xs, ys, hs_t, ns_t,
               ox, oy, oh, on, od, idx_v, rows_v, dval_v, sem):
    wid = lax.axis_index("s") * _NC + lax.axis_index("c")
    base = wid * BPW

    pltpu.sync_copy(xs.at[pl.ds(base, BPW)], idx_v)
    pltpu.async_copy(table.at[idx_v], rows_v, sem).wait()
    pltpu.sync_copy(rows_v, ox.at[pl.ds(base, BPW)])
    pltpu.async_copy(dflat.at[idx_v], dval_v, sem).wait()
    pltpu.sync_copy(dval_v, od.at[pl.ds(base, BPW)])

    pltpu.sync_copy(ys.at[pl.ds(base, BPW)], idx_v)
    pltpu.async_copy(table.at[idx_v], rows_v, sem).wait()
    pltpu.sync_copy(rows_v, oy.at[pl.ds(base, BPW)])

    def h_body(h, _):
        pltpu.sync_copy(hs_t.at[h, pl.ds(base, BPW)], idx_v)
        pltpu.async_copy(table.at[idx_v], rows_v, sem).wait()
        pltpu.sync_copy(rows_v, oh.at[h, pl.ds(base, BPW)])
        return _
    lax.fori_loop(0, H, h_body, 0)

    def n_body(n, _):
        pltpu.sync_copy(ns_t.at[n, pl.ds(base, BPW)], idx_v)
        pltpu.async_copy(table.at[idx_v], rows_v, sem).wait()
        pltpu.sync_copy(rows_v, on.at[n, pl.ds(base, BPW)])
        return _
    lax.fori_loop(0, N, n_body, 0)


BB = 512


def _logsig(z):
    return jnp.minimum(z, 0.0) - jnp.log1p(jnp.exp(-jnp.abs(z)))


def _tc_body(x_ref, y_ref, h_ref, n_ref, dlt_ref, et_ref, ht_ref, mk_ref,
             out_ref):
    x = x_ref[...]
    y = y_ref[...]
    dlt = dlt_ref[...]
    et = et_ref[...]

    p_mu = -jnp.sum(jnp.square(x - y), axis=1)

    alphas = []
    for h in range(H):
        hh = h_ref[h]
        alphas.append(-jnp.sum(jnp.square(x - hh), axis=1))

    m = alphas[0]
    for h in range(1, H):
        m = jnp.maximum(m, alphas[h])
    exps = [jnp.exp(a - m) for a in alphas]
    ssum = exps[0]
    for h in range(1, H):
        ssum = ssum + exps[h]

    A = jnp.zeros_like(p_mu)
    Wsum = jnp.zeros_like(p_mu)
    S = jnp.zeros_like(p_mu)
    hw = jnp.zeros_like(x)
    for h in range(H):
        hh = h_ref[h]
        dt = jnp.abs(et - ht_ref[h])
        decay = jnp.exp(dlt * dt) * mk_ref[h]
        w = (exps[h] / ssum) * decay
        A = A + w * alphas[h]
        Wsum = Wsum + w
        S = S + w * jnp.sum(hh * hh, axis=1)
        hw = hw + w[:, None] * hh

    loss = _logsig(p_mu + A)
    for n in range(N):
        nn = n_ref[n]
        n_mu = -jnp.sum(jnp.square(x - nn), axis=1)
        sn = jnp.sum(nn * nn, axis=1)
        dot = jnp.sum(hw * nn, axis=1)
        loss = loss - _logsig(n_mu - S - Wsum * sn + 2.0 * dot)
    out_ref[...] = loss


def kernel(xs, ys, e_times, hs, h_times, neg_node, h_times_mask,
           emb_table, delta_table):
    xs = xs.astype(jnp.int32)
    ys = ys.astype(jnp.int32)
    hs_t = jnp.transpose(hs).astype(jnp.int32)
    ns_t = jnp.transpose(neg_node).astype(jnp.int32)
    ht_t = jnp.transpose(h_times)
    mk_t = jnp.transpose(h_times_mask)
    dflat = delta_table.reshape(NODE)

    ox, oy, oh, on, od = _sc_gather(emb_table, dflat, xs, ys, hs_t, ns_t)

    grid = B // BB
    loss = pl.pallas_call(
        _tc_body,
        grid=(grid,),
        in_specs=[
            pl.BlockSpec((BB, D), lambda i: (i, 0)),
            pl.BlockSpec((BB, D), lambda i: (i, 0)),
            pl.BlockSpec((H, BB, D), lambda i: (0, i, 0)),
            pl.BlockSpec((N, BB, D), lambda i: (0, i, 0)),
            pl.BlockSpec((BB,), lambda i: (i,)),
            pl.BlockSpec((BB,), lambda i: (i,)),
            pl.BlockSpec((H, BB), lambda i: (0, i)),
            pl.BlockSpec((H, BB), lambda i: (0, i)),
        ],
        out_specs=pl.BlockSpec((BB,), lambda i: (i,)),
        out_shape=jax.ShapeDtypeStruct((B,), jnp.float32),
    )(ox, oy, oh, on, od, e_times, ht_t, mk_t)
    return loss


# speedup vs baseline: 1.4066x; 1.0262x over previous
import functools

import jax
import jax.numpy as jnp
from jax import lax
from jax.experimental import pallas as pl
from jax.experimental.pallas import tpu as pltpu
from jax.experimental.pallas import tpu_sc as plsc

NODE = 1000000
D = 64
B = 16384
H = 20
N = 5
B2 = B // 2

_info = plsc.get_sparse_core_info()
_NC, _NS = _info.num_cores, _info.num_subcores
NW = _NC * _NS
RPW = B2 // NW

_sc_mesh = plsc.VectorSubcoreMesh(core_axis_name="c", subcore_axis_name="s")


@functools.partial(
    pl.kernel,
    mesh=_sc_mesh,
    compiler_params=pltpu.CompilerParams(use_tc_tiling_on_sc=False),
    out_type=[
        jax.ShapeDtypeStruct((B2, 2 * D), jnp.float32),
        jax.ShapeDtypeStruct((B2, 2 * D), jnp.float32),
        jax.ShapeDtypeStruct((H, B2, 2 * D), jnp.float32),
        jax.ShapeDtypeStruct((N, B2, 2 * D), jnp.float32),
        jax.ShapeDtypeStruct((B,), jnp.float32),
    ],
    scratch_types=[
        pltpu.VMEM((RPW,), jnp.int32),
        pltpu.VMEM((RPW, D), jnp.float32),
        pltpu.VMEM((RPW,), jnp.float32),
        pltpu.SemaphoreType.DMA,
    ],
)
def _sc_gather(table, dflat, xs, ys, hs_t, ns_t,
               ox, oy, oh, on, od, idx_v, rows_v, dval_v, sem):
    wid = lax.axis_index("s") * _NC + lax.axis_index("c")
    rbase = wid * RPW

    for s in range(2):
        ebase = s * B2 + rbase
        col = pl.ds(s * D, D)

        pltpu.sync_copy(xs.at[pl.ds(ebase, RPW)], idx_v)
        pltpu.async_copy(table.at[idx_v], rows_v, sem).wait()
        pltpu.sync_copy(rows_v, ox.at[pl.ds(rbase, RPW), col])
        pltpu.async_copy(dflat.at[idx_v], dval_v, sem).wait()
        pltpu.sync_copy(dval_v, od.at[pl.ds(ebase, RPW)])

        pltpu.sync_copy(ys.at[pl.ds(ebase, RPW)], idx_v)
        pltpu.async_copy(table.at[idx_v], rows_v, sem).wait()
        pltpu.sync_copy(rows_v, oy.at[pl.ds(rbase, RPW), col])

        def h_body(h, _):
            pltpu.sync_copy(hs_t.at[h, pl.ds(ebase, RPW)], idx_v)
            pltpu.async_copy(table.at[idx_v], rows_v, sem).wait()
            pltpu.sync_copy(rows_v, oh.at[h, pl.ds(rbase, RPW), col])
            return _
        lax.fori_loop(0, H, h_body, 0)

        def n_body(n, _):
            pltpu.sync_copy(ns_t.at[n, pl.ds(ebase, RPW)], idx_v)
            pltpu.async_copy(table.at[idx_v], rows_v, sem).wait()
            pltpu.sync_copy(rows_v, on.at[n, pl.ds(rbase, RPW), col])
            return _
        lax.fori_loop(0, N, n_body, 0)


BBR = 256


def _logsig(z):
    return jnp.minimum(z, 0.0) - jnp.log1p(jnp.exp(-jnp.abs(z)))


def _half_sums(z):
    return jnp.sum(z[:, :D], axis=1), jnp.sum(z[:, D:], axis=1)


def _tc_body(x_ref, y_ref, h_ref, n_ref, dlt_ref, et_ref, ht_ref, mk_ref,
             out_ref):
    x = x_ref[...]
    y = y_ref[...]
    dlt = (dlt_ref[0], dlt_ref[1])
    et = (et_ref[0], et_ref[1])

    p_mu = _half_sums(jnp.square(x - y))
    p_mu = (-p_mu[0], -p_mu[1])

    alphas = []
    for h in range(H):
        slo, shi = _half_sums(jnp.square(x - h_ref[h]))
        alphas.append((-slo, -shi))

    m = [alphas[0][0], alphas[0][1]]
    for h in range(1, H):
        m[0] = jnp.maximum(m[0], alphas[h][0])
        m[1] = jnp.maximum(m[1], alphas[h][1])
    exps = [(jnp.exp(a[0] - m[0]), jnp.exp(a[1] - m[1])) for a in alphas]
    ssum = [exps[0][0], exps[0][1]]
    for h in range(1, H):
        ssum[0] = ssum[0] + exps[h][0]
        ssum[1] = ssum[1] + exps[h][1]

    zero = jnp.zeros_like(p_mu[0])
    A = [zero, zero]
    Wsum = [zero, zero]
    S = [zero, zero]
    hw = jnp.zeros_like(x)
    for h in range(H):
        hh = h_ref[h]
        sh = _half_sums(hh * hh)
        w128_parts = []
        for s in range(2):
            dt = jnp.abs(et[s] - ht_ref[h, s])
            decay = jnp.exp(dlt[s] * dt) * mk_ref[h, s]
            w = (exps[h][s] / ssum[s]) * decay
            A[s] = A[s] + w * alphas[h][s]
            Wsum[s] = Wsum[s] + w
            S[s] = S[s] + w * sh[s]
            w128_parts.append(jnp.broadcast_to(w[:, None], (BBR, D)))
        hw = hw + jnp.concatenate(w128_parts, axis=1) * hh

    loss = [_logsig(p_mu[0] + A[0]), _logsig(p_mu[1] + A[1])]
    for n in range(N):
        nn = n_ref[n]
        nmu = _half_sums(jnp.square(x - nn))
        sn = _half_sums(nn * nn)
        dots = _half_sums(hw * nn)
        for s in range(2):
            n_lam = -nmu[s] - S[s] - Wsum[s] * sn[s] + 2.0 * dots[s]
            loss[s] = loss[s] - _logsig(n_lam)
    out_ref[0] = loss[0]
    out_ref[1] = loss[1]


def kernel(xs, ys, e_times, hs, h_times, neg_node, h_times_mask,
           emb_table, delta_table):
    xs = xs.astype(jnp.int32)
    ys = ys.astype(jnp.int32)
    hs_t = jnp.transpose(hs).astype(jnp.int32)
    ns_t = jnp.transpose(neg_node).astype(jnp.int32)
    ht3 = jnp.transpose(h_times).reshape(H, 2, B2)
    mk3 = jnp.transpose(h_times_mask).reshape(H, 2, B2)
    et2 = e_times.reshape(2, B2)
    dflat = delta_table.reshape(NODE)

    ox, oy, oh, on, od = _sc_gather(emb_table, dflat, xs, ys, hs_t, ns_t)
    od2 = od.reshape(2, B2)

    grid = B2 // BBR
    loss2 = pl.pallas_call(
        _tc_body,
        grid=(grid,),
        in_specs=[
            pl.BlockSpec((BBR, 2 * D), lambda i: (i, 0)),
            pl.BlockSpec((BBR, 2 * D), lambda i: (i, 0)),
            pl.BlockSpec((H, BBR, 2 * D), lambda i: (0, i, 0)),
            pl.BlockSpec((N, BBR, 2 * D), lambda i: (0, i, 0)),
            pl.BlockSpec((2, BBR), lambda i: (0, i)),
            pl.BlockSpec((2, BBR), lambda i: (0, i)),
            pl.BlockSpec((H, 2, BBR), lambda i: (0, 0, i)),
            pl.BlockSpec((H, 2, BBR), lambda i: (0, 0, i)),
        ],
        out_specs=pl.BlockSpec((2, BBR), lambda i: (0, i)),
        out_shape=jax.ShapeDtypeStruct((2, B2), jnp.float32),
    )(ox, oy, oh, on, od2, et2, ht3, mk3)
    return loss2.reshape(B)


# speedup vs baseline: 1.9130x; 1.3601x over previous
import functools

import jax
import jax.numpy as jnp
from jax import lax
from jax.experimental import pallas as pl
from jax.experimental.pallas import tpu as pltpu
from jax.experimental.pallas import tpu_sc as plsc

NODE = 1000000
D = 64
B = 16384
H = 20
N = 5
B2 = B // 2

_info = plsc.get_sparse_core_info()
_NC, _NS = _info.num_cores, _info.num_subcores
NW = _NC * _NS
RPW = B2 // NW

_sc_mesh = plsc.VectorSubcoreMesh(core_axis_name="c", subcore_axis_name="s")


@functools.partial(
    pl.kernel,
    mesh=_sc_mesh,
    compiler_params=pltpu.CompilerParams(use_tc_tiling_on_sc=False),
    out_type=[
        jax.ShapeDtypeStruct((B2, 2 * D), jnp.float32),
        jax.ShapeDtypeStruct((B2, 2 * D), jnp.float32),
        jax.ShapeDtypeStruct((H, B2, 2 * D), jnp.float32),
        jax.ShapeDtypeStruct((N, B2, 2 * D), jnp.float32),
        jax.ShapeDtypeStruct((B,), jnp.float32),
    ],
    scratch_types=[
        pltpu.VMEM((RPW,), jnp.int32),
        pltpu.VMEM((RPW, D), jnp.float32),
        pltpu.VMEM((RPW,), jnp.float32),
        pltpu.SemaphoreType.DMA,
    ],
)
def _sc_gather(table, dflat, xs, ys, hs_t, ns_t,
               ox, oy, oh, on, od, idx_v, rows_v, dval_v, sem):
    wid = lax.axis_index("s") * _NC + lax.axis_index("c")
    rbase = wid * RPW

    for s in range(2):
        ebase = s * B2 + rbase
        col = pl.ds(s * D, D)

        pltpu.sync_copy(xs.at[pl.ds(ebase, RPW)], idx_v)
        pltpu.async_copy(table.at[idx_v], rows_v, sem).wait()
        pltpu.sync_copy(rows_v, ox.at[pl.ds(rbase, RPW), col])
        pltpu.async_copy(dflat.at[idx_v], dval_v, sem).wait()
        pltpu.sync_copy(dval_v, od.at[pl.ds(ebase, RPW)])

        pltpu.sync_copy(ys.at[pl.ds(ebase, RPW)], idx_v)
        pltpu.async_copy(table.at[idx_v], rows_v, sem).wait()
        pltpu.sync_copy(rows_v, oy.at[pl.ds(rbase, RPW), col])

        def h_body(h, _):
            pltpu.sync_copy(hs_t.at[h, pl.ds(ebase, RPW)], idx_v)
            pltpu.async_copy(table.at[idx_v], rows_v, sem).wait()
            pltpu.sync_copy(rows_v, oh.at[h, pl.ds(rbase, RPW), col])
            return _
        lax.fori_loop(0, H, h_body, 0)

        def n_body(n, _):
            pltpu.sync_copy(ns_t.at[n, pl.ds(ebase, RPW)], idx_v)
            pltpu.async_copy(table.at[idx_v], rows_v, sem).wait()
            pltpu.sync_copy(rows_v, on.at[n, pl.ds(rbase, RPW), col])
            return _
        lax.fori_loop(0, N, n_body, 0)


BBR = 512


def _logsig(z):
    return jnp.minimum(z, 0.0) - jnp.log1p(jnp.exp(-jnp.abs(z)))


def _dot(a, b):
    return jax.lax.dot(a, b, preferred_element_type=jnp.float32)


def _tc_body(x_ref, y_ref, h_ref, n_ref, tp1_ref, tp2_ref, etb_ref, dltb_ref,
             selh_ref, selht_ref, fullsel_ref, sumsel_ref, out_ref):
    x = x_ref[...]
    y = y_ref[...]
    fullsel = fullsel_ref[...]
    sumsel = sumsel_ref[...]

    d = x - y
    p_mu = -_dot(d * d, fullsel)

    alpha_c = jnp.zeros_like(x)
    sh_c = jnp.zeros_like(x)
    for h in range(H):
        hh = h_ref[h]
        d = x - hh
        alpha_c = alpha_c + _dot(d * d, selh_ref[h])
        sh_c = sh_c + _dot(hh * hh, selh_ref[h])
    alpha_c = -alpha_c

    lane = jax.lax.broadcasted_iota(jnp.int32, (BBR, 2 * D), 1)
    valid = jnp.logical_or(lane < H, jnp.logical_and(lane >= D, lane < D + H))
    exps_c = jnp.where(valid, jnp.exp(alpha_c), 0.0)
    ssum = jnp.maximum(_dot(exps_c, sumsel), 1e-35)

    dt_c = jnp.abs(etb_ref[...] - tp1_ref[...])
    decay_c = jnp.exp(dltb_ref[...] * dt_c) * tp2_ref[...]
    w_c = exps_c * decay_c / ssum

    A = _dot(w_c * alpha_c, sumsel)
    Wsum = _dot(w_c, sumsel)
    S = _dot(w_c * sh_c, sumsel)
    hw = jnp.zeros_like(x)
    for h in range(H):
        hw = hw + _dot(w_c, selht_ref[h]) * h_ref[h]

    loss = _logsig(p_mu + A)
    for n in range(N):
        nn = n_ref[n]
        d = x - nn
        n_mu = -_dot(d * d, fullsel)
        sn = _dot(nn * nn, fullsel)
        dot = _dot(hw * nn, fullsel)
        loss = loss - _logsig(n_mu - S - Wsum * sn + 2.0 * dot)
    out_ref[...] = loss


def kernel(xs, ys, e_times, hs, h_times, neg_node, h_times_mask,
           emb_table, delta_table):
    xs = xs.astype(jnp.int32)
    ys = ys.astype(jnp.int32)
    hs_t = jnp.transpose(hs).astype(jnp.int32)
    ns_t = jnp.transpose(neg_node).astype(jnp.int32)
    dflat = delta_table.reshape(NODE)

    pad = jnp.zeros((B2, D - H), jnp.float32)
    tp1 = jnp.concatenate([h_times[:B2], pad, h_times[B2:], pad], axis=1)
    tp2 = jnp.concatenate(
        [h_times_mask[:B2], pad, h_times_mask[B2:], pad], axis=1)

    def _bcast2(v):
        return jnp.concatenate(
            [jnp.broadcast_to(v[:B2, None], (B2, D)),
             jnp.broadcast_to(v[B2:, None], (B2, D))], axis=1)

    etb = _bcast2(e_times)

    ii = jax.lax.broadcasted_iota(jnp.int32, (2 * D, 2 * D), 0)
    jj = jax.lax.broadcasted_iota(jnp.int32, (2 * D, 2 * D), 1)
    ilo, jlo = ii < D, jj < D
    fullsel = (ilo == jlo).astype(jnp.float32)
    sumsel = (jnp.where(ilo, ii, 999) < H) & jlo
    sumsel = (sumsel | (((jnp.where(ilo, -1, ii - D)) < H)
                        & (jnp.where(ilo, -1, ii - D) >= 0) & ~jlo))
    sumsel = sumsel.astype(jnp.float32)
    hh_idx = jax.lax.broadcasted_iota(jnp.int32, (H, 2 * D, 2 * D), 0)
    hi2 = jax.lax.broadcasted_iota(jnp.int32, (H, 2 * D, 2 * D), 1)
    hj2 = jax.lax.broadcasted_iota(jnp.int32, (H, 2 * D, 2 * D), 2)
    selh = (((hi2 < D) & (hj2 == hh_idx))
            | ((hi2 >= D) & (hj2 == hh_idx + D))).astype(jnp.float32)
    selht = jnp.transpose(selh, (0, 2, 1))

    ox, oy, oh, on, od = _sc_gather(emb_table, dflat, xs, ys, hs_t, ns_t)
    dltb = _bcast2(od)

    grid = B2 // BBR
    cst2 = pl.BlockSpec((2 * D, 2 * D), lambda i: (0, 0))
    cst3 = pl.BlockSpec((H, 2 * D, 2 * D), lambda i: (0, 0, 0))
    row2 = pl.BlockSpec((BBR, 2 * D), lambda i: (i, 0))
    losspack = pl.pallas_call(
        _tc_body,
        grid=(grid,),
        in_specs=[
            row2, row2,
            pl.BlockSpec((H, BBR, 2 * D), lambda i: (0, i, 0)),
            pl.BlockSpec((N, BBR, 2 * D), lambda i: (0, i, 0)),
            row2, row2, row2, row2,
            cst3, cst3, cst2, cst2,
        ],
        out_specs=row2,
        out_shape=jax.ShapeDtypeStruct((B2, 2 * D), jnp.float32),
    )(ox, oy, oh, on, tp1, tp2, etb, dltb, selh, selht, fullsel, sumsel)
    return jnp.concatenate([losspack[:, 0], losspack[:, D]], axis=0)


# speedup vs baseline: 1.9499x; 1.0193x over previous
import functools

import jax
import jax.numpy as jnp
from jax import lax
from jax.experimental import pallas as pl
from jax.experimental.pallas import tpu as pltpu
from jax.experimental.pallas import tpu_sc as plsc

NODE = 1000000
D = 64
B = 16384
H = 20
N = 5
B2 = B // 2

_info = plsc.get_sparse_core_info()
_NC, _NS = _info.num_cores, _info.num_subcores
NW = _NC * _NS
RPW = B2 // NW

_sc_mesh = plsc.VectorSubcoreMesh(core_axis_name="c", subcore_axis_name="s")


@functools.partial(
    pl.kernel,
    mesh=_sc_mesh,
    compiler_params=pltpu.CompilerParams(use_tc_tiling_on_sc=False),
    out_type=[
        jax.ShapeDtypeStruct((B2, 2 * D), jnp.float32),
        jax.ShapeDtypeStruct((B2, 2 * D), jnp.float32),
        jax.ShapeDtypeStruct((H, B2, 2 * D), jnp.float32),
        jax.ShapeDtypeStruct((N, B2, 2 * D), jnp.float32),
    ],
    scratch_types=[
        pltpu.VMEM((RPW,), jnp.int32),
        pltpu.VMEM((RPW, D), jnp.float32),
        pltpu.SemaphoreType.DMA,
    ],
)
def _sc_gather(table, xs, ys, hs_t, ns_t,
               ox, oy, oh, on, idx_v, rows_v, sem):
    wid = lax.axis_index("s") * _NC + lax.axis_index("c")
    rbase = wid * RPW

    for s in range(2):
        ebase = s * B2 + rbase
        col = pl.ds(s * D, D)

        pltpu.sync_copy(xs.at[pl.ds(ebase, RPW)], idx_v)
        pltpu.async_copy(table.at[idx_v], rows_v, sem).wait()
        pltpu.sync_copy(rows_v, ox.at[pl.ds(rbase, RPW), col])

        pltpu.sync_copy(ys.at[pl.ds(ebase, RPW)], idx_v)
        pltpu.async_copy(table.at[idx_v], rows_v, sem).wait()
        pltpu.sync_copy(rows_v, oy.at[pl.ds(rbase, RPW), col])

        def h_body(h, _):
            pltpu.sync_copy(hs_t.at[h, pl.ds(ebase, RPW)], idx_v)
            pltpu.async_copy(table.at[idx_v], rows_v, sem).wait()
            pltpu.sync_copy(rows_v, oh.at[h, pl.ds(rbase, RPW), col])
            return _
        lax.fori_loop(0, H, h_body, 0)

        def n_body(n, _):
            pltpu.sync_copy(ns_t.at[n, pl.ds(ebase, RPW)], idx_v)
            pltpu.async_copy(table.at[idx_v], rows_v, sem).wait()
            pltpu.sync_copy(rows_v, on.at[n, pl.ds(rbase, RPW), col])
            return _
        lax.fori_loop(0, N, n_body, 0)


BPW = B // NW


@functools.partial(
    pl.kernel,
    mesh=_sc_mesh,
    compiler_params=pltpu.CompilerParams(use_tc_tiling_on_sc=False),
    out_type=jax.ShapeDtypeStruct((B,), jnp.float32),
    scratch_types=[
        pltpu.VMEM((BPW,), jnp.int32),
        pltpu.VMEM((BPW,), jnp.float32),
        pltpu.SemaphoreType.DMA,
    ],
)
def _sc_delta(dflat, xs, od, idx_v, dval_v, sem):
    wid = lax.axis_index("s") * _NC + lax.axis_index("c")
    base = wid * BPW
    pltpu.sync_copy(xs.at[pl.ds(base, BPW)], idx_v)
    pltpu.async_copy(dflat.at[idx_v], dval_v, sem).wait()
    pltpu.sync_copy(dval_v, od.at[pl.ds(base, BPW)])


BBR = 512


def _logsig(z):
    return jnp.minimum(z, 0.0) - jnp.log1p(jnp.exp(-jnp.abs(z)))


def _dot(a, b):
    return jax.lax.dot(a, b, preferred_element_type=jnp.float32)


def _tc_body(x_ref, y_ref, h_ref, n_ref, tp1_ref, tp2_ref, etb_ref, dltb_ref,
             selh_ref, selht_ref, fullsel_ref, sumsel_ref, out_ref):
    x = x_ref[...]
    y = y_ref[...]
    fullsel = fullsel_ref[...]
    sumsel = sumsel_ref[...]

    d = x - y
    p_mu = -_dot(d * d, fullsel)

    alpha_c = jnp.zeros_like(x)
    sh_c = jnp.zeros_like(x)
    for h in range(H):
        hh = h_ref[h]
        d = x - hh
        alpha_c = alpha_c + _dot(d * d, selh_ref[h])
        sh_c = sh_c + _dot(hh * hh, selh_ref[h])
    alpha_c = -alpha_c

    lane = jax.lax.broadcasted_iota(jnp.int32, (BBR, 2 * D), 1)
    valid = jnp.logical_or(lane < H, jnp.logical_and(lane >= D, lane < D + H))
    exps_c = jnp.where(valid, jnp.exp(alpha_c), 0.0)
    ssum = jnp.maximum(_dot(exps_c, sumsel), 1e-35)

    dt_c = jnp.abs(etb_ref[...] - tp1_ref[...])
    decay_c = jnp.exp(dltb_ref[...] * dt_c) * tp2_ref[...]
    w_c = exps_c * decay_c / ssum

    A = _dot(w_c * alpha_c, sumsel)
    Wsum = _dot(w_c, sumsel)
    S = _dot(w_c * sh_c, sumsel)
    hw = jnp.zeros_like(x)
    for h in range(H):
        hw = hw + _dot(w_c, selht_ref[h]) * h_ref[h]

    loss = _logsig(p_mu + A)
    for n in range(N):
        nn = n_ref[n]
        d = x - nn
        n_mu = -_dot(d * d, fullsel)
        sn = _dot(nn * nn, fullsel)
        dot = _dot(hw * nn, fullsel)
        loss = loss - _logsig(n_mu - S - Wsum * sn + 2.0 * dot)
    out_ref[...] = loss


def kernel(xs, ys, e_times, hs, h_times, neg_node, h_times_mask,
           emb_table, delta_table):
    xs = xs.astype(jnp.int32)
    ys = ys.astype(jnp.int32)
    hs_t = jnp.transpose(hs).astype(jnp.int32)
    ns_t = jnp.transpose(neg_node).astype(jnp.int32)

    pad = jnp.zeros((B2, D - H), jnp.float32)
    tp1 = jnp.concatenate([h_times[:B2], pad, h_times[B2:], pad], axis=1)
    tp2 = jnp.concatenate(
        [h_times_mask[:B2], pad, h_times_mask[B2:], pad], axis=1)

    def _bcast2(v):
        return jnp.concatenate(
            [jnp.broadcast_to(v[:B2, None], (B2, D)),
             jnp.broadcast_to(v[B2:, None], (B2, D))], axis=1)

    etb = _bcast2(e_times)

    ii = jax.lax.broadcasted_iota(jnp.int32, (2 * D, 2 * D), 0)
    jj = jax.lax.broadcasted_iota(jnp.int32, (2 * D, 2 * D), 1)
    ilo, jlo = ii < D, jj < D
    fullsel = (ilo == jlo).astype(jnp.float32)
    sumsel = (jnp.where(ilo, ii, 999) < H) & jlo
    sumsel = (sumsel | (((jnp.where(ilo, -1, ii - D)) < H)
                        & (jnp.where(ilo, -1, ii - D) >= 0) & ~jlo))
    sumsel = sumsel.astype(jnp.float32)
    hh_idx = jax.lax.broadcasted_iota(jnp.int32, (H, 2 * D, 2 * D), 0)
    hi2 = jax.lax.broadcasted_iota(jnp.int32, (H, 2 * D, 2 * D), 1)
    hj2 = jax.lax.broadcasted_iota(jnp.int32, (H, 2 * D, 2 * D), 2)
    selh = (((hi2 < D) & (hj2 == hh_idx))
            | ((hi2 >= D) & (hj2 == hh_idx + D))).astype(jnp.float32)
    selht = jnp.transpose(selh, (0, 2, 1))

    ox, oy, oh, on = _sc_gather(emb_table, xs, ys, hs_t, ns_t)
    dflat = delta_table.reshape(NODE)
    od = _sc_delta(dflat, xs)
    dltb = _bcast2(od)

    grid = B2 // BBR
    cst2 = pl.BlockSpec((2 * D, 2 * D), lambda i: (0, 0))
    cst3 = pl.BlockSpec((H, 2 * D, 2 * D), lambda i: (0, 0, 0))
    row2 = pl.BlockSpec((BBR, 2 * D), lambda i: (i, 0))
    losspack = pl.pallas_call(
        _tc_body,
        grid=(grid,),
        in_specs=[
            row2, row2,
            pl.BlockSpec((H, BBR, 2 * D), lambda i: (0, i, 0)),
            pl.BlockSpec((N, BBR, 2 * D), lambda i: (0, i, 0)),
            row2, row2, row2, row2,
            cst3, cst3, cst2, cst2,
        ],
        out_specs=row2,
        out_shape=jax.ShapeDtypeStruct((B2, 2 * D), jnp.float32),
    )(ox, oy, oh, on, tp1, tp2, etb, dltb, selh, selht, fullsel, sumsel)
    return jnp.concatenate([losspack[:, 0], losspack[:, D]], axis=0)
